# baseline (speedup 1.0000x reference)
import functools

import jax
import jax.numpy as jnp
from jax import lax
from jax.experimental import pallas as pl
from jax.experimental.pallas import tpu as pltpu
from jax.experimental.pallas import tpu_sc as plsc

N = 10000
E = 320000
D = 128
C = 40

NC = 2
NS = 16
NW = NC * NS
CHUNK = 128
CP = 48
NBUF = 4
K = ((-(-E // (NW * CHUNK)) + NBUF - 1) // NBUF) * NBUF
E_PAD = NW * CHUNK * K
N_ACC = ((N + CHUNK + NW * 8 - 1) // (NW * 8)) * (NW * 8)
RPS = N_ACC // NS
BLK = 128
NBLK = RPS // BLK

_mesh = plsc.VectorSubcoreMesh(
    core_axis_name="c", subcore_axis_name="s", num_cores=NC, num_subcores=NS
)
_sc_params = pltpu.CompilerParams(
    use_tc_tiling_on_sc=False, needs_layout_passes=False
)


def _rsqrt16(x):
    i = plsc.bitcast(x, jnp.int32)
    y = plsc.bitcast(jnp.int32(0x5F3759DF) - (i >> 1), jnp.float32)
    for _ in range(3):
        y = y * (1.5 - 0.5 * x * y * y)
    return y


@functools.partial(
    pl.kernel,
    out_type=jax.ShapeDtypeStruct((NC, N_ACC, 16), jnp.float32),
    mesh=_mesh,
    scratch_types=[
        pltpu.VMEM((K, CHUNK), jnp.int32),
        pltpu.VMEM((CHUNK, 16), jnp.float32),
        pltpu.VMEM_SHARED((N_ACC, 16), jnp.float32),
    ],
    compiler_params=_sc_params,
)
def _deg_kernel(rows_hbm, zeros16_hbm, ones16_hbm, out_hbm, rowv, onesv, acc):
    c = lax.axis_index("c")
    s = lax.axis_index("s")
    wid = c * NS + s
    pltpu.sync_copy(rows_hbm.at[wid], rowv)
    pltpu.sync_copy(ones16_hbm, onesv)
    pltpu.sync_copy(zeros16_hbm.at[pl.ds(s * RPS, RPS)], acc.at[pl.ds(s * RPS, RPS)])
    plsc.subcore_barrier()

    @pl.loop(0, K)
    def _(k):
        pltpu.sync_copy(onesv, acc.at[rowv.at[k]], add=True)

    plsc.subcore_barrier()
    pltpu.sync_copy(acc.at[pl.ds(s * RPS, RPS)], out_hbm.at[c, pl.ds(s * RPS, RPS)])


def _make_prop(first_step):
    @functools.partial(
        pl.kernel,
        out_type=jax.ShapeDtypeStruct((NC, N_ACC, CP), jnp.float32),
        mesh=_mesh,
        scratch_types=[
            pltpu.VMEM((K, CHUNK), jnp.int32),
            pltpu.VMEM((K, CHUNK), jnp.int32),
            pltpu.VMEM((NBUF, CHUNK, CP), jnp.float32),
            pltpu.VMEM((BLK, 16), jnp.float32),
            pltpu.VMEM((BLK, 16), jnp.float32),
            pltpu.VMEM((BLK, CP), jnp.float32),
            pltpu.VMEM((BLK, CP), jnp.float32),
            pltpu.VMEM((BLK, CP), jnp.float32),
            pltpu.VMEM_SHARED((N_ACC, CP), jnp.float32),
            pltpu.VMEM_SHARED((N_ACC, CP), jnp.float32),
            pltpu.SemaphoreType.DMA((NBUF,)),
        ],
        compiler_params=_sc_params,
    )
    def _prop(y_hbm, cnt_hbm, zeros48_hbm, rows_hbm, cols_hbm, out_hbm,
              rowv, colv, gbuf, c0v, c1v, av, bv, zv, zsh, acc, gsem):
        c = lax.axis_index("c")
        s = lax.axis_index("s")
        wid = c * NS + s
        pltpu.sync_copy(rows_hbm.at[wid], rowv)
        pltpu.sync_copy(cols_hbm.at[wid], colv)

        @pl.loop(0, NBLK)
        def _(blk):
            r0 = s * RPS + blk * BLK
            pltpu.sync_copy(cnt_hbm.at[0, pl.ds(r0, BLK)], c0v)
            pltpu.sync_copy(cnt_hbm.at[1, pl.ds(r0, BLK)], c1v)
            if first_step:
                pltpu.sync_copy(y_hbm.at[pl.ds(r0, BLK)], av)
            else:
                pltpu.sync_copy(y_hbm.at[0, pl.ds(r0, BLK)], av)
                pltpu.sync_copy(y_hbm.at[1, pl.ds(r0, BLK)], bv)

            @pl.loop(0, BLK)
            def _(i):
                deg = 1.0 + c0v[i, :] + c1v[i, :]
                if first_step:
                    scale = _rsqrt16(deg)
                else:
                    scale = 1.0 / deg
                for g in range(CP // 16):
                    sl = (i, pl.ds(g * 16, 16))
                    if first_step:
                        zv[sl] = av[sl] * scale
                    else:
                        zv[sl] = (av[sl] + bv[sl]) * scale

            pltpu.sync_copy(zv, zsh.at[pl.ds(r0, BLK)])

            @pl.when(c == 0)
            def _():
                pltpu.sync_copy(zv, acc.at[pl.ds(r0, BLK)])

            @pl.when(c == 1)
            def _():
                pltpu.sync_copy(zeros48_hbm.at[pl.ds(r0, BLK)],
                                acc.at[pl.ds(r0, BLK)])

        plsc.subcore_barrier()

        for b in range(NBUF):
            pltpu.async_copy(zsh.at[colv.at[b]], gbuf.at[b], gsem.at[b])

        @pl.loop(0, K // NBUF - 1)
        def _(g):
            for b in range(NBUF):
                k = g * NBUF + b
                pltpu.make_async_copy(zsh.at[colv.at[k]], gbuf.at[b],
                                      gsem.at[b]).wait()
                pltpu.sync_copy(gbuf.at[b], acc.at[rowv.at[k]], add=True)
                pltpu.async_copy(zsh.at[colv.at[k + NBUF]], gbuf.at[b],
                                 gsem.at[b])

        for b in range(NBUF):
            k = K - NBUF + b
            pltpu.make_async_copy(zsh.at[colv.at[k]], gbuf.at[b],
                                  gsem.at[b]).wait()
            pltpu.sync_copy(gbuf.at[b], acc.at[rowv.at[k]], add=True)

        plsc.subcore_barrier()
        pltpu.sync_copy(acc.at[pl.ds(s * RPS, RPS)],
                        out_hbm.at[c, pl.ds(s * RPS, RPS)])

    return _prop


_prop1 = _make_prop(True)
_prop2 = _make_prop(False)


@functools.partial(
    pl.kernel,
    out_type=jax.ShapeDtypeStruct((N_ACC, CP), jnp.float32),
    mesh=_mesh,
    scratch_types=[
        pltpu.VMEM((N_ACC // NW, 16), jnp.float32),
        pltpu.VMEM((N_ACC // NW, 16), jnp.float32),
        pltpu.VMEM((N_ACC // NW, CP), jnp.float32),
        pltpu.VMEM((N_ACC // NW, CP), jnp.float32),
        pltpu.VMEM((N_ACC // NW, CP), jnp.float32),
        pltpu.VMEM((1, CP), jnp.float32),
    ],
    compiler_params=_sc_params,
)
def _fin_kernel(t_hbm, cnt_hbm, b_hbm, out_hbm, c0v, c1v, av, bv, zv, biasv):
    c = lax.axis_index("c")
    s = lax.axis_index("s")
    wid = c * NS + s
    pltpu.sync_copy(b_hbm, biasv)
    rps = N_ACC // NW
    r0 = wid * rps
    pltpu.sync_copy(cnt_hbm.at[0, pl.ds(r0, rps)], c0v)
    pltpu.sync_copy(cnt_hbm.at[1, pl.ds(r0, rps)], c1v)
    pltpu.sync_copy(t_hbm.at[0, pl.ds(r0, rps)], av)
    pltpu.sync_copy(t_hbm.at[1, pl.ds(r0, rps)], bv)

    @pl.loop(0, rps)
    def _(i):
        deg = 1.0 + c0v[i, :] + c1v[i, :]
        r = _rsqrt16(deg)
        for g in range(CP // 16):
            sl = (i, pl.ds(g * 16, 16))
            zv[sl] = (av[sl] + bv[sl]) * r + biasv[0, pl.ds(g * 16, 16)]

    pltpu.sync_copy(zv, out_hbm.at[pl.ds(r0, rps)])


def _mm_body(x_ref, w_ref, z_ref):
    z_ref[pl.ds(0, N), :] = jnp.dot(
        x_ref[...], w_ref[...], preferred_element_type=jnp.float32
    )


def _tc_matmul(x, wp):
    return pl.pallas_call(
        _mm_body,
        out_shape=jax.ShapeDtypeStruct((N_ACC, CP), jnp.float32),
    )(x, wp)


@jax.jit
def kernel(X, edge_index, W, b):
    ei = edge_index.astype(jnp.int32)
    pad = E_PAD - E
    padv = jax.lax.iota(jnp.int32, pad) % CHUNK
    rows = jnp.concatenate([ei[:, 0], N + padv]).reshape(NW, K, CHUNK)
    cols = jnp.concatenate([ei[:, 1], padv]).reshape(NW, K, CHUNK)

    wp = jnp.zeros((D, CP), jnp.float32).at[:, :C].set(W.T)
    bp = jnp.zeros((1, CP), jnp.float32).at[0, :C].set(b)

    zeros16 = jnp.zeros((N_ACC, 16), jnp.float32)
    zeros48 = jnp.zeros((N_ACC, CP), jnp.float32)
    ones16 = jnp.ones((CHUNK, 16), jnp.float32)

    cnt = _deg_kernel(rows, zeros16, ones16)
    y0 = _tc_matmul(X, wp)
    t1 = _prop1(y0, cnt, zeros48, rows, cols)
    t2 = _prop2(t1, cnt, zeros48, rows, cols)
    out = _fin_kernel(t2, cnt, bp)
    return out[:N, :C]


# speedup vs baseline: 1.2558x; 1.2558x over previous
import functools

import jax
import jax.numpy as jnp
from jax import lax
from jax.experimental import pallas as pl
from jax.experimental.pallas import tpu as pltpu
from jax.experimental.pallas import tpu_sc as plsc

N = 10000
E = 320000
D = 128
C = 40

NC = 2
NS = 16
NW = NC * NS
CHUNK = 128
CP = 48
NBUF = 4
K = ((-(-E // (NW * CHUNK)) + NBUF - 1) // NBUF) * NBUF
E_PAD = NW * CHUNK * K
N_ACC = ((N + CHUNK + NW * 8 - 1) // (NW * 8)) * (NW * 8)
RPS = N_ACC // NS
BLK = 128
NBLK = RPS // BLK

_mesh = plsc.VectorSubcoreMesh(
    core_axis_name="c", subcore_axis_name="s", num_cores=NC, num_subcores=NS
)
_sc_params = pltpu.CompilerParams(
    use_tc_tiling_on_sc=False, needs_layout_passes=False
)


def _rsqrt16(x):
    i = plsc.bitcast(x, jnp.int32)
    y = plsc.bitcast(jnp.int32(0x5F3759DF) - (i >> 1), jnp.float32)
    for _ in range(2):
        y = y * (1.5 - 0.5 * x * y * y)
    return y


@functools.partial(
    pl.kernel,
    out_type=jax.ShapeDtypeStruct((NC, N_ACC, 16), jnp.float32),
    mesh=_mesh,
    scratch_types=[
        pltpu.VMEM((K, CHUNK), jnp.int32),
        pltpu.VMEM((CHUNK, 16), jnp.float32),
        pltpu.VMEM_SHARED((N_ACC, 16), jnp.float32),
    ],
    compiler_params=_sc_params,
)
def _deg_kernel(rows_hbm, zeros16_hbm, ones16_hbm, out_hbm, rowv, onesv, acc):
    c = lax.axis_index("c")
    s = lax.axis_index("s")
    wid = c * NS + s
    pltpu.sync_copy(rows_hbm.at[wid], rowv)
    pltpu.sync_copy(ones16_hbm, onesv)
    pltpu.sync_copy(zeros16_hbm.at[pl.ds(s * RPS, RPS)], acc.at[pl.ds(s * RPS, RPS)])
    plsc.subcore_barrier()

    @pl.loop(0, K)
    def _(k):
        pltpu.sync_copy(onesv, acc.at[rowv.at[k]], add=True)

    plsc.subcore_barrier()
    pltpu.sync_copy(acc.at[pl.ds(s * RPS, RPS)], out_hbm.at[c, pl.ds(s * RPS, RPS)])


def _make_prop(first_step):
    ablk = RPS if first_step else RPS // 2
    nblk = RPS // ablk

    @functools.partial(
        pl.kernel,
        out_type=(
            jax.ShapeDtypeStruct((NC, N_ACC, CP), jnp.float32),
            jax.ShapeDtypeStruct((NC, N_ACC, CP), jnp.float32),
        ),
        mesh=_mesh,
        scratch_types=[
            pltpu.VMEM((K, CHUNK), jnp.int32),
            pltpu.VMEM((K, CHUNK), jnp.int32),
            pltpu.VMEM((NBUF, CHUNK, CP), jnp.float32),
            pltpu.VMEM((ablk, 16), jnp.float32),
            pltpu.VMEM((ablk, 16), jnp.float32),
            pltpu.VMEM((ablk, CP), jnp.float32),
            pltpu.VMEM((ablk, CP), jnp.float32),
            pltpu.VMEM_SHARED((N_ACC, CP), jnp.float32),
            pltpu.SemaphoreType.DMA((NBUF,)),
        ],
        compiler_params=_sc_params,
    )
    def _prop(y_hbm, cnt_hbm, zeros48_hbm, rows_hbm, cols_hbm, out_hbm,
              z_hbm, rowv, colv, gbuf, c0v, c1v, av, bv, acc, gsem):
        c = lax.axis_index("c")
        s = lax.axis_index("s")
        wid = c * NS + s
        pltpu.sync_copy(rows_hbm.at[wid], rowv)
        pltpu.sync_copy(cols_hbm.at[wid], colv)

        @pl.loop(0, nblk)
        def _(blk):
            r0 = s * RPS + blk * ablk
            pltpu.sync_copy(cnt_hbm.at[0, pl.ds(r0, ablk)], c0v)
            pltpu.sync_copy(cnt_hbm.at[1, pl.ds(r0, ablk)], c1v)
            if first_step:
                pltpu.sync_copy(y_hbm.at[pl.ds(r0, ablk)], av)
            else:
                pltpu.sync_copy(y_hbm.at[0, pl.ds(r0, ablk)], av)
                pltpu.sync_copy(y_hbm.at[1, pl.ds(r0, ablk)], bv)

            @pl.loop(0, ablk)
            def _(i):
                deg = 1.0 + c0v[i, :] + c1v[i, :]
                if first_step:
                    scale = _rsqrt16(deg)
                else:
                    scale = 1.0 / deg
                for g in range(CP // 16):
                    sl = (i, pl.ds(g * 16, 16))
                    if first_step:
                        av[sl] = av[sl] * scale
                    else:
                        av[sl] = (av[sl] + bv[sl]) * scale

            pltpu.sync_copy(av, z_hbm.at[c, pl.ds(r0, ablk)])

            @pl.when(c == 0)
            def _():
                pltpu.sync_copy(av, acc.at[pl.ds(r0, ablk)])

            @pl.when(c == 1)
            def _():
                pltpu.sync_copy(zeros48_hbm.at[pl.ds(r0, ablk)],
                                acc.at[pl.ds(r0, ablk)])

        plsc.subcore_barrier()
        zc = z_hbm.at[c]

        for b in range(NBUF):
            pltpu.async_copy(zc.at[colv.at[b]], gbuf.at[b], gsem.at[b])

        @pl.loop(0, K // NBUF - 1)
        def _(g):
            for b in range(NBUF):
                k = g * NBUF + b
                pltpu.make_async_copy(zc.at[colv.at[k]], gbuf.at[b],
                                      gsem.at[b]).wait()
                pltpu.sync_copy(gbuf.at[b], acc.at[rowv.at[k]], add=True)
                pltpu.async_copy(zc.at[colv.at[k + NBUF]], gbuf.at[b],
                                 gsem.at[b])

        for b in range(NBUF):
            k = K - NBUF + b
            pltpu.make_async_copy(zc.at[colv.at[k]], gbuf.at[b], gsem.at[b]).wait()
            pltpu.sync_copy(gbuf.at[b], acc.at[rowv.at[k]], add=True)

        plsc.subcore_barrier()
        pltpu.sync_copy(acc.at[pl.ds(s * RPS, RPS)],
                        out_hbm.at[c, pl.ds(s * RPS, RPS)])

    return _prop


_prop1 = _make_prop(True)
_prop2 = _make_prop(False)


@functools.partial(
    pl.kernel,
    out_type=jax.ShapeDtypeStruct((N_ACC, CP), jnp.float32),
    mesh=_mesh,
    scratch_types=[
        pltpu.VMEM((N_ACC // NW, 16), jnp.float32),
        pltpu.VMEM((N_ACC // NW, 16), jnp.float32),
        pltpu.VMEM((N_ACC // NW, CP), jnp.float32),
        pltpu.VMEM((N_ACC // NW, CP), jnp.float32),
        pltpu.VMEM((N_ACC // NW, CP), jnp.float32),
        pltpu.VMEM((1, CP), jnp.float32),
    ],
    compiler_params=_sc_params,
)
def _fin_kernel(t_hbm, cnt_hbm, b_hbm, out_hbm, c0v, c1v, av, bv, zv, biasv):
    c = lax.axis_index("c")
    s = lax.axis_index("s")
    wid = c * NS + s
    pltpu.sync_copy(b_hbm, biasv)
    rps = N_ACC // NW
    r0 = wid * rps
    pltpu.sync_copy(cnt_hbm.at[0, pl.ds(r0, rps)], c0v)
    pltpu.sync_copy(cnt_hbm.at[1, pl.ds(r0, rps)], c1v)
    pltpu.sync_copy(t_hbm.at[0, pl.ds(r0, rps)], av)
    pltpu.sync_copy(t_hbm.at[1, pl.ds(r0, rps)], bv)

    @pl.loop(0, rps)
    def _(i):
        deg = 1.0 + c0v[i, :] + c1v[i, :]
        r = _rsqrt16(deg)
        for g in range(CP // 16):
            sl = (i, pl.ds(g * 16, 16))
            zv[sl] = (av[sl] + bv[sl]) * r + biasv[0, pl.ds(g * 16, 16)]

    pltpu.sync_copy(zv, out_hbm.at[pl.ds(r0, rps)])


def _mm_body(x_ref, w_ref, z_ref):
    z_ref[pl.ds(0, N), :] = jnp.dot(
        x_ref[...], w_ref[...], preferred_element_type=jnp.float32
    )


def _tc_matmul(x, wp):
    return pl.pallas_call(
        _mm_body,
        out_shape=jax.ShapeDtypeStruct((N_ACC, CP), jnp.float32),
    )(x, wp)


@jax.jit
def kernel(X, edge_index, W, b):
    ei = edge_index.astype(jnp.int32)
    pad = E_PAD - E
    padv = jax.lax.iota(jnp.int32, pad) % CHUNK
    rows = jnp.concatenate([ei[:, 0], N + padv]).reshape(NW, K, CHUNK)
    cols = jnp.concatenate([ei[:, 1], padv]).reshape(NW, K, CHUNK)

    wp = jnp.zeros((D, CP), jnp.float32).at[:, :C].set(W.T)
    bp = jnp.zeros((1, CP), jnp.float32).at[0, :C].set(b)

    zeros16 = jnp.zeros((N_ACC, 16), jnp.float32)
    zeros48 = jnp.zeros((N_ACC, CP), jnp.float32)
    ones16 = jnp.ones((CHUNK, 16), jnp.float32)

    cnt = _deg_kernel(rows, zeros16, ones16)
    y0 = _tc_matmul(X, wp)
    t1, _ = _prop1(y0, cnt, zeros48, rows, cols)
    t2, _ = _prop2(t1, cnt, zeros48, rows, cols)
    out = _fin_kernel(t2, cnt, bp)
    return out[:N, :C]


# speedup vs baseline: 1.2677x; 1.0095x over previous
import functools

import jax
import jax.numpy as jnp
from jax import lax
from jax.experimental import pallas as pl
from jax.experimental.pallas import tpu as pltpu
from jax.experimental.pallas import tpu_sc as plsc

N = 10000
E = 320000
D = 128
C = 40

NC = 2
NS = 16
NW = NC * NS
CHUNK = 128
CP = 48
NBUF = 4
K = ((-(-E // (NW * CHUNK)) + NBUF - 1) // NBUF) * NBUF
E_PAD = NW * CHUNK * K
N_ACC = ((N + CHUNK + NW * 8 - 1) // (NW * 8)) * (NW * 8)
RPS = N_ACC // NS
BLK = 128
NBLK = RPS // BLK

_mesh = plsc.VectorSubcoreMesh(
    core_axis_name="c", subcore_axis_name="s", num_cores=NC, num_subcores=NS
)
_sc_params = pltpu.CompilerParams(
    use_tc_tiling_on_sc=False, needs_layout_passes=False
)


def _rsqrt16(x):
    i = plsc.bitcast(x, jnp.int32)
    y = plsc.bitcast(jnp.int32(0x5F3759DF) - (i >> 1), jnp.float32)
    for _ in range(2):
        y = y * (1.5 - 0.5 * x * y * y)
    return y


@functools.partial(
    pl.kernel,
    out_type=jax.ShapeDtypeStruct((NC, N_ACC, 16), jnp.float32),
    mesh=_mesh,
    scratch_types=[
        pltpu.VMEM((K, CHUNK), jnp.int32),
        pltpu.VMEM((CHUNK, 16), jnp.float32),
        pltpu.VMEM_SHARED((N_ACC, 16), jnp.float32),
    ],
    compiler_params=_sc_params,
)
def _deg_kernel(rows_hbm, zeros16_hbm, ones16_hbm, out_hbm, rowv, onesv, acc):
    c = lax.axis_index("c")
    s = lax.axis_index("s")
    wid = c * NS + s
    pltpu.sync_copy(rows_hbm.at[wid], rowv)
    pltpu.sync_copy(ones16_hbm, onesv)
    pltpu.sync_copy(zeros16_hbm.at[pl.ds(s * RPS, RPS)], acc.at[pl.ds(s * RPS, RPS)])
    plsc.subcore_barrier()

    @pl.loop(0, K)
    def _(k):
        pltpu.sync_copy(onesv, acc.at[rowv.at[k]], add=True)

    plsc.subcore_barrier()
    pltpu.sync_copy(acc.at[pl.ds(s * RPS, RPS)], out_hbm.at[c, pl.ds(s * RPS, RPS)])


def _make_prop(first_step):
    ablk = RPS if first_step else RPS // 2
    nblk = RPS // ablk
    bshape = (ablk, CP) if not first_step else (8, CP)

    @functools.partial(
        pl.kernel,
        out_type=(
            jax.ShapeDtypeStruct((NC, N_ACC, CP), jnp.float32),
            jax.ShapeDtypeStruct((NC, N_ACC, CP), jnp.float32),
        ),
        mesh=_mesh,
        scratch_types=[
            pltpu.VMEM((K, CHUNK), jnp.int32),
            pltpu.VMEM((K, CHUNK), jnp.int32),
            pltpu.VMEM((NBUF, CHUNK, CP), jnp.float32),
            pltpu.VMEM((ablk, 16), jnp.float32),
            pltpu.VMEM((ablk, 16), jnp.float32),
            pltpu.VMEM((ablk, CP), jnp.float32),
            pltpu.VMEM(bshape, jnp.float32),
            pltpu.VMEM_SHARED((N_ACC, CP), jnp.float32),
            pltpu.SemaphoreType.DMA((NBUF,)),
        ],
        compiler_params=_sc_params,
    )
    def _prop(y_hbm, cnt_hbm, zeros48_hbm, rows_hbm, cols_hbm, out_hbm,
              z_hbm, rowv, colv, gbuf, c0v, c1v, av, bv, acc, gsem):
        c = lax.axis_index("c")
        s = lax.axis_index("s")
        wid = c * NS + s
        pltpu.sync_copy(rows_hbm.at[wid], rowv)
        pltpu.sync_copy(cols_hbm.at[wid], colv)

        @pl.loop(0, nblk)
        def _(blk):
            r0 = s * RPS + blk * ablk
            pltpu.sync_copy(cnt_hbm.at[0, pl.ds(r0, ablk)], c0v)
            pltpu.sync_copy(cnt_hbm.at[1, pl.ds(r0, ablk)], c1v)
            if first_step:
                pltpu.sync_copy(y_hbm.at[pl.ds(r0, ablk)], av)
            else:
                pltpu.sync_copy(y_hbm.at[0, pl.ds(r0, ablk)], av)
                pltpu.sync_copy(y_hbm.at[1, pl.ds(r0, ablk)], bv)

            @pl.loop(0, ablk)
            def _(i):
                deg = 1.0 + c0v[i, :] + c1v[i, :]
                if first_step:
                    scale = _rsqrt16(deg)
                else:
                    scale = 1.0 / deg
                for g in range(CP // 16):
                    sl = (i, pl.ds(g * 16, 16))
                    if first_step:
                        av[sl] = av[sl] * scale
                    else:
                        av[sl] = (av[sl] + bv[sl]) * scale

            pltpu.sync_copy(av, z_hbm.at[c, pl.ds(r0, ablk)])

            @pl.when(c == 0)
            def _():
                pltpu.sync_copy(av, acc.at[pl.ds(r0, ablk)])

            @pl.when(c == 1)
            def _():
                pltpu.sync_copy(zeros48_hbm.at[pl.ds(r0, ablk)],
                                acc.at[pl.ds(r0, ablk)])

        plsc.subcore_barrier()
        zc = z_hbm.at[c]

        for b in range(NBUF):
            pltpu.async_copy(zc.at[colv.at[b]], gbuf.at[b], gsem.at[b])

        @pl.loop(0, K // NBUF - 1)
        def _(g):
            for b in range(NBUF):
                k = g * NBUF + b
                pltpu.make_async_copy(zc.at[colv.at[k]], gbuf.at[b],
                                      gsem.at[b]).wait()
                pltpu.sync_copy(gbuf.at[b], acc.at[rowv.at[k]], add=True)
                pltpu.async_copy(zc.at[colv.at[k + NBUF]], gbuf.at[b],
                                 gsem.at[b])

        for b in range(NBUF):
            k = K - NBUF + b
            pltpu.make_async_copy(zc.at[colv.at[k]], gbuf.at[b], gsem.at[b]).wait()
            pltpu.sync_copy(gbuf.at[b], acc.at[rowv.at[k]], add=True)

        plsc.subcore_barrier()
        pltpu.sync_copy(acc.at[pl.ds(s * RPS, RPS)],
                        out_hbm.at[c, pl.ds(s * RPS, RPS)])

    return _prop


_prop1 = _make_prop(True)
_prop2 = _make_prop(False)


@functools.partial(
    pl.kernel,
    out_type=jax.ShapeDtypeStruct((N, C), jnp.float32),
    mesh=_mesh,
    scratch_types=[
        pltpu.VMEM((N_ACC // NW, 16), jnp.float32),
        pltpu.VMEM((N_ACC // NW, 16), jnp.float32),
        pltpu.VMEM((N_ACC // NW, CP), jnp.float32),
        pltpu.VMEM((N_ACC // NW, CP), jnp.float32),
        pltpu.VMEM((N_ACC // NW, CP), jnp.float32),
        pltpu.VMEM((1, CP), jnp.float32),
    ],
    compiler_params=_sc_params,
)
def _fin_kernel(t_hbm, cnt_hbm, b_hbm, out_hbm, c0v, c1v, av, bv, zv, biasv):
    c = lax.axis_index("c")
    s = lax.axis_index("s")
    wid = c * NS + s
    pltpu.sync_copy(b_hbm, biasv)
    rps = N_ACC // NW
    r0 = wid * rps
    pltpu.sync_copy(cnt_hbm.at[0, pl.ds(r0, rps)], c0v)
    pltpu.sync_copy(cnt_hbm.at[1, pl.ds(r0, rps)], c1v)
    pltpu.sync_copy(t_hbm.at[0, pl.ds(r0, rps)], av)
    pltpu.sync_copy(t_hbm.at[1, pl.ds(r0, rps)], bv)

    @pl.loop(0, rps)
    def _(i):
        deg = 1.0 + c0v[i, :] + c1v[i, :]
        r = _rsqrt16(deg)
        for g in range(CP // 16):
            sl = (i, pl.ds(g * 16, 16))
            zv[sl] = (av[sl] + bv[sl]) * r + biasv[0, pl.ds(g * 16, 16)]

    nrem = N - (NW - 1) * rps

    @pl.when(wid < NW - 1)
    def _():
        pltpu.sync_copy(zv.at[:, pl.ds(0, C)], out_hbm.at[pl.ds(r0, rps)])

    @pl.when(wid == NW - 1)
    def _():
        pltpu.sync_copy(zv.at[pl.ds(0, nrem), pl.ds(0, C)],
                        out_hbm.at[pl.ds(r0, nrem)])


def _mm_body(x_ref, w_ref, z_ref):
    z_ref[pl.ds(0, N), :] = jnp.dot(
        x_ref[...], w_ref[...], preferred_element_type=jnp.float32
    )


def _tc_matmul(x, wp):
    return pl.pallas_call(
        _mm_body,
        out_shape=jax.ShapeDtypeStruct((N_ACC, CP), jnp.float32),
    )(x, wp)


@jax.jit
def kernel(X, edge_index, W, b):
    ei = edge_index.astype(jnp.int32).T
    pad = E_PAD - E
    padv = jax.lax.iota(jnp.int32, pad) % CHUNK
    rows = jnp.concatenate([ei[0], N + padv]).reshape(NW, K, CHUNK)
    cols = jnp.concatenate([ei[1], padv]).reshape(NW, K, CHUNK)

    wp = jnp.zeros((D, CP), jnp.float32).at[:, :C].set(W.T)
    bp = jnp.zeros((1, CP), jnp.float32).at[0, :C].set(b)

    zeros16 = jnp.zeros((N_ACC, 16), jnp.float32)
    zeros48 = jnp.zeros((N_ACC, CP), jnp.float32)
    ones16 = jnp.ones((CHUNK, 16), jnp.float32)

    cnt = _deg_kernel(rows, zeros16, ones16)
    y0 = _tc_matmul(X, wp)
    t1, _ = _prop1(y0, cnt, zeros48, rows, cols)
    t2, _ = _prop2(t1, cnt, zeros48, rows, cols)
    return _fin_kernel(t2, cnt, bp)


# speedup vs baseline: 1.3366x; 1.0544x over previous
import functools

import jax
import jax.numpy as jnp
from jax import lax
from jax.experimental import pallas as pl
from jax.experimental.pallas import tpu as pltpu
from jax.experimental.pallas import tpu_sc as plsc

N = 10000
E = 320000
D = 128
C = 40

NC = 2
NS = 16
NW = NC * NS
CHUNK = 128
CP = 48
NBUF = 4
K = ((-(-E // (NW * CHUNK)) + NBUF - 1) // NBUF) * NBUF
E_PAD = NW * CHUNK * K
N_ACC = ((N + CHUNK + NW * 8 - 1) // (NW * 8)) * (NW * 8)
RPS = N_ACC // NS
BLK = 128
NBLK = RPS // BLK

_mesh = plsc.VectorSubcoreMesh(
    core_axis_name="c", subcore_axis_name="s", num_cores=NC, num_subcores=NS
)
_sc_params = pltpu.CompilerParams(
    use_tc_tiling_on_sc=False, needs_layout_passes=False
)


def _rsqrt16(x):
    i = plsc.bitcast(x, jnp.int32)
    y = plsc.bitcast(jnp.int32(0x5F3759DF) - (i >> 1), jnp.float32)
    for _ in range(2):
        y = y * (1.5 - 0.5 * x * y * y)
    return y


@functools.partial(
    pl.kernel,
    out_type=jax.ShapeDtypeStruct((NC, N_ACC, 16), jnp.float32),
    mesh=_mesh,
    scratch_types=[
        pltpu.VMEM((K, CHUNK), jnp.int32),
        pltpu.VMEM((CHUNK, 16), jnp.float32),
        pltpu.VMEM_SHARED((N_ACC, 16), jnp.float32),
    ],
    compiler_params=_sc_params,
)
def _deg_kernel(rows_hbm, zeros16_hbm, ones16_hbm, out_hbm, rowv, onesv, acc):
    c = lax.axis_index("c")
    s = lax.axis_index("s")
    wid = c * NS + s
    pltpu.sync_copy(rows_hbm.at[wid], rowv)
    pltpu.sync_copy(ones16_hbm, onesv)
    pltpu.sync_copy(zeros16_hbm.at[pl.ds(s * RPS, RPS)], acc.at[pl.ds(s * RPS, RPS)])
    plsc.subcore_barrier()

    @pl.loop(0, K)
    def _(k):
        pltpu.sync_copy(onesv, acc.at[rowv.at[k]], add=True)

    plsc.subcore_barrier()
    pltpu.sync_copy(acc.at[pl.ds(s * RPS, RPS)], out_hbm.at[c, pl.ds(s * RPS, RPS)])


def _make_prop(first_step):
    cshape = (RPS, 16) if first_step else (8, 16)
    bshape = (8, CP) if first_step else (RPS // 2, CP)

    @functools.partial(
        pl.kernel,
        out_type=(
            jax.ShapeDtypeStruct((NC, N_ACC, CP), jnp.float32),
            jax.ShapeDtypeStruct((NC, N_ACC, CP), jnp.float32),
        ),
        mesh=_mesh,
        scratch_types=[
            pltpu.VMEM((K, CHUNK), jnp.int32),
            pltpu.VMEM((K, CHUNK), jnp.int32),
            pltpu.VMEM((NBUF, CHUNK, CP), jnp.float32),
            pltpu.VMEM(cshape, jnp.float32),
            pltpu.VMEM(cshape, jnp.float32),
            pltpu.VMEM((RPS, CP), jnp.float32),
            pltpu.VMEM(bshape, jnp.float32),
            pltpu.VMEM_SHARED((N_ACC, CP), jnp.float32),
            pltpu.SemaphoreType.DMA((NBUF,)),
        ],
        compiler_params=_sc_params,
    )
    def _prop(y_hbm, cnt_hbm, zeros48_hbm, rows_hbm, cols_hbm, out_hbm,
              z_hbm, rowv, colv, gbuf, c0v, c1v, av, bv, acc, gsem):
        c = lax.axis_index("c")
        s = lax.axis_index("s")
        wid = c * NS + s
        pltpu.sync_copy(rows_hbm.at[wid], rowv)
        pltpu.sync_copy(cols_hbm.at[wid], colv)

        r0 = s * RPS
        if first_step:
            pltpu.async_copy(cnt_hbm.at[0, pl.ds(r0, RPS)], c0v, gsem.at[0])
            pltpu.async_copy(cnt_hbm.at[1, pl.ds(r0, RPS)], c1v, gsem.at[1])
            pltpu.async_copy(y_hbm.at[pl.ds(r0, RPS)], av, gsem.at[2])
            pltpu.make_async_copy(cnt_hbm.at[0, pl.ds(r0, RPS)], c0v,
                                  gsem.at[0]).wait()
            pltpu.make_async_copy(cnt_hbm.at[1, pl.ds(r0, RPS)], c1v,
                                  gsem.at[1]).wait()
            pltpu.make_async_copy(y_hbm.at[pl.ds(r0, RPS)], av,
                                  gsem.at[2]).wait()
        else:
            pltpu.async_copy(y_hbm.at[0, pl.ds(r0, RPS)], av, gsem.at[0])
            pltpu.async_copy(y_hbm.at[1, pl.ds(r0, RPS // 2)], bv, gsem.at[1])
            pltpu.make_async_copy(y_hbm.at[0, pl.ds(r0, RPS)], av,
                                  gsem.at[0]).wait()
            pltpu.make_async_copy(y_hbm.at[1, pl.ds(r0, RPS // 2)], bv,
                                  gsem.at[1]).wait()

        if first_step:

            @pl.loop(0, RPS)
            def _(i):
                scale = _rsqrt16(1.0 + c0v[i, :] + c1v[i, :])
                for g in range(CP // 16):
                    sl = (i, pl.ds(g * 16, 16))
                    av[sl] = av[sl] * scale

        else:
            for half in range(2):
                h0 = half * (RPS // 2)
                if half == 1:
                    pltpu.sync_copy(
                        y_hbm.at[1, pl.ds(r0 + h0, RPS // 2)], bv)

                @pl.loop(0, RPS // 2)
                def _(i):
                    for g in range(CP // 16):
                        av[(h0 + i, pl.ds(g * 16, 16))] = (
                            av[(h0 + i, pl.ds(g * 16, 16))]
                            + bv[(i, pl.ds(g * 16, 16))])

        pltpu.sync_copy(av, z_hbm.at[c, pl.ds(r0, RPS)])

        @pl.when(c == 0)
        def _():
            pltpu.sync_copy(av, acc.at[pl.ds(r0, RPS)])

        @pl.when(c == 1)
        def _():
            pltpu.sync_copy(zeros48_hbm.at[pl.ds(r0, RPS)],
                            acc.at[pl.ds(r0, RPS)])

        plsc.subcore_barrier()
        zc = z_hbm.at[c]

        for b in range(NBUF):
            pltpu.async_copy(zc.at[colv.at[b]], gbuf.at[b], gsem.at[b])

        @pl.loop(0, K // NBUF - 1)
        def _(g):
            for b in range(NBUF):
                k = g * NBUF + b
                pltpu.make_async_copy(zc.at[colv.at[k]], gbuf.at[b],
                                      gsem.at[b]).wait()
                pltpu.sync_copy(gbuf.at[b], acc.at[rowv.at[k]], add=True)
                pltpu.async_copy(zc.at[colv.at[k + NBUF]], gbuf.at[b],
                                 gsem.at[b])

        for b in range(NBUF):
            k = K - NBUF + b
            pltpu.make_async_copy(zc.at[colv.at[k]], gbuf.at[b], gsem.at[b]).wait()
            pltpu.sync_copy(gbuf.at[b], acc.at[rowv.at[k]], add=True)

        plsc.subcore_barrier()
        if first_step:
            pltpu.sync_copy(acc.at[pl.ds(r0, RPS)], av)

            @pl.loop(0, RPS)
            def _(i):
                dinv = 1.0 / (1.0 + c0v[i, :] + c1v[i, :])
                for g in range(CP // 16):
                    sl = (i, pl.ds(g * 16, 16))
                    av[sl] = av[sl] * dinv

            pltpu.sync_copy(av, out_hbm.at[c, pl.ds(r0, RPS)])
        else:
            pltpu.sync_copy(acc.at[pl.ds(r0, RPS)],
                            out_hbm.at[c, pl.ds(r0, RPS)])

    return _prop


_prop1 = _make_prop(True)
_prop2 = _make_prop(False)


@functools.partial(
    pl.kernel,
    out_type=jax.ShapeDtypeStruct((N, C), jnp.float32),
    mesh=_mesh,
    scratch_types=[
        pltpu.VMEM((N_ACC // NW, 16), jnp.float32),
        pltpu.VMEM((N_ACC // NW, 16), jnp.float32),
        pltpu.VMEM((N_ACC // NW, CP), jnp.float32),
        pltpu.VMEM((N_ACC // NW, CP), jnp.float32),
        pltpu.VMEM((N_ACC // NW, CP), jnp.float32),
        pltpu.VMEM((1, CP), jnp.float32),
    ],
    compiler_params=_sc_params,
)
def _fin_kernel(t_hbm, cnt_hbm, b_hbm, out_hbm, c0v, c1v, av, bv, zv, biasv):
    c = lax.axis_index("c")
    s = lax.axis_index("s")
    wid = c * NS + s
    pltpu.sync_copy(b_hbm, biasv)
    rps = N_ACC // NW
    r0 = wid * rps
    pltpu.sync_copy(cnt_hbm.at[0, pl.ds(r0, rps)], c0v)
    pltpu.sync_copy(cnt_hbm.at[1, pl.ds(r0, rps)], c1v)
    pltpu.sync_copy(t_hbm.at[0, pl.ds(r0, rps)], av)
    pltpu.sync_copy(t_hbm.at[1, pl.ds(r0, rps)], bv)

    @pl.loop(0, rps)
    def _(i):
        deg = 1.0 + c0v[i, :] + c1v[i, :]
        r = _rsqrt16(deg)
        for g in range(CP // 16):
            sl = (i, pl.ds(g * 16, 16))
            zv[sl] = (av[sl] + bv[sl]) * r + biasv[0, pl.ds(g * 16, 16)]

    nrem = N - (NW - 1) * rps

    @pl.when(wid < NW - 1)
    def _():
        pltpu.sync_copy(zv.at[:, pl.ds(0, C)], out_hbm.at[pl.ds(r0, rps)])

    @pl.when(wid == NW - 1)
    def _():
        pltpu.sync_copy(zv.at[pl.ds(0, nrem), pl.ds(0, C)],
                        out_hbm.at[pl.ds(r0, nrem)])


def _mm_body(x_ref, w_ref, z_ref):
    z_ref[pl.ds(0, N), :] = jnp.dot(
        x_ref[...], w_ref[...], preferred_element_type=jnp.float32
    )


def _tc_matmul(x, wp):
    return pl.pallas_call(
        _mm_body,
        out_shape=jax.ShapeDtypeStruct((N_ACC, CP), jnp.float32),
    )(x, wp)


@jax.jit
def kernel(X, edge_index, W, b):
    ei = edge_index.astype(jnp.int32).T
    pad = E_PAD - E
    padv = jax.lax.iota(jnp.int32, pad) % CHUNK
    rows = jnp.concatenate([ei[0], N + padv]).reshape(NW, K, CHUNK)
    cols = jnp.concatenate([ei[1], padv]).reshape(NW, K, CHUNK)

    wp = jnp.zeros((D, CP), jnp.float32).at[:, :C].set(W.T)
    bp = jnp.zeros((1, CP), jnp.float32).at[0, :C].set(b)

    zeros16 = jnp.zeros((N_ACC, 16), jnp.float32)
    zeros48 = jnp.zeros((N_ACC, CP), jnp.float32)
    ones16 = jnp.ones((CHUNK, 16), jnp.float32)

    cnt = _deg_kernel(rows, zeros16, ones16)
    y0 = _tc_matmul(X, wp)
    t1, _ = _prop1(y0, cnt, zeros48, rows, cols)
    t2, _ = _prop2(t1, cnt, zeros48, rows, cols)
    return _fin_kernel(t2, cnt, bp)
